# baseline (speedup 1.0000x reference)
import functools

import jax
import jax.numpy as jnp
from jax import lax
from jax.experimental import pallas as pl
from jax.experimental.pallas import tpu as pltpu
from jax.experimental.pallas import tpu_sc as plsc

N, D, E, TOP_K = 8192, 2048, 16, 2
BLOCK = 1024

L = 16
NW = 32
CHUNK = N // NW
GROUPS = CHUNK // L


def _noisy_body(x_ref, w_ref, b_ref, eps_ref, noisy_ref):
    acc = jnp.dot(x_ref[...], w_ref[...], preferred_element_type=jnp.float32)
    acc = acc + b_ref[...]
    logits = acc[:, :E]
    nlog = acc[:, E:]
    noisy_ref[...] = logits + eps_ref[...] * jax.nn.softplus(nlog)


def _route_body(noisy_hbm, sparse_hbm, idx_hbm, noisy_v, out_v, idx_v):
    wid = lax.axis_index("s") * 2 + lax.axis_index("c")
    base = wid * CHUNK
    pltpu.sync_copy(noisy_hbm.at[pl.ds(base, CHUNK)], noisy_v)

    lanes = lax.iota(jnp.int32, L)
    zero_i = jnp.full((L,), 0, jnp.int32)
    one_f = jnp.full((L,), 1.0, jnp.float32)
    zero_f = jnp.full((L,), 0.0, jnp.float32)

    def group(g, carry):
        toks = lanes + jnp.full((L,), g * L, jnp.int32)
        m1 = plsc.load_gather(noisy_v, [toks, zero_i])
        i1 = zero_i
        m2 = jnp.full((L,), -jnp.inf, jnp.float32)
        i2 = zero_i
        for e in range(1, E):
            es = jnp.full((L,), e, jnp.int32)
            ve = plsc.load_gather(noisy_v, [toks, es])
            gt1 = ve > m1
            gt2 = ve > m2
            m2 = jnp.where(gt1, m1, jnp.where(gt2, ve, m2))
            i2 = jnp.where(gt1, i1, jnp.where(gt2, es, i2))
            m1 = jnp.where(gt1, ve, m1)
            i1 = jnp.where(gt1, es, i1)
        e2 = jnp.exp(m2 - m1)
        p1 = one_f / (one_f + e2)
        p2 = e2 * p1
        for e in range(E):
            es = jnp.full((L,), e, jnp.int32)
            col = jnp.where(i1 == es, p1, jnp.where(i2 == es, p2, zero_f))
            plsc.store_scatter(out_v, [toks, es], col)
        plsc.store_scatter(idx_v, [toks, zero_i], i1)
        plsc.store_scatter(idx_v, [toks, zero_i + 1], i2)
        return carry

    lax.fori_loop(0, GROUPS, group, 0)

    pltpu.sync_copy(out_v, sparse_hbm.at[pl.ds(base, CHUNK)])
    pltpu.sync_copy(idx_v, idx_hbm.at[pl.ds(base, CHUNK)])


@jax.jit
def kernel(x, Wg, bg, Wn, bn, eps):
    noisy = pl.pallas_call(
        _noisy_body,
        grid=(N // BLOCK,),
        in_specs=[
            pl.BlockSpec((BLOCK, D), lambda i: (i, 0)),
            pl.BlockSpec((D, 2 * E), lambda i: (0, 0)),
            pl.BlockSpec((1, 2 * E), lambda i: (0, 0)),
            pl.BlockSpec((BLOCK, E), lambda i: (i, 0)),
        ],
        out_specs=pl.BlockSpec((BLOCK, E), lambda i: (i, 0)),
        out_shape=jax.ShapeDtypeStruct((N, E), jnp.float32),
    )(x, jnp.concatenate([Wg, Wn], axis=1),
      jnp.concatenate([bg, bn]).reshape(1, 2 * E), eps)

    route = pl.kernel(
        _route_body,
        out_type=(
            jax.ShapeDtypeStruct((N, E), jnp.float32),
            jax.ShapeDtypeStruct((N, TOP_K), jnp.int32),
        ),
        mesh=plsc.VectorSubcoreMesh(core_axis_name="c", subcore_axis_name="s"),
        compiler_params=pltpu.CompilerParams(
            needs_layout_passes=False,
            disable_bounds_checks=True,
            disable_semaphore_checks=True,
        ),
        scratch_types=[
            pltpu.VMEM((CHUNK, E), jnp.float32),
            pltpu.VMEM((CHUNK, E), jnp.float32),
            pltpu.VMEM((CHUNK, TOP_K), jnp.int32),
        ],
    )
    sparse, idx = route(noisy)
    return sparse, idx


# speedup vs baseline: 1.0673x; 1.0673x over previous
import functools

import jax
import jax.numpy as jnp
from jax import lax
from jax.experimental import pallas as pl
from jax.experimental.pallas import tpu as pltpu
from jax.experimental.pallas import tpu_sc as plsc

N, D, E, TOP_K = 8192, 2048, 16, 2
BLOCK = 1024

L = 16
NW = 32
CHUNK = N // NW
GROUPS = CHUNK // L


def _noisy_body(x_ref, w_ref, b_ref, eps_ref, noisy_ref):
    acc = jnp.dot(x_ref[...], w_ref[...], preferred_element_type=jnp.float32)
    acc = acc + b_ref[...]
    logits = acc[:, :E]
    nlog = acc[:, E:]
    noisy_ref[...] = logits + eps_ref[...] * jax.nn.softplus(nlog)


HALF = CHUNK // 2


def _route_body(noisy_hbm, sparse_hbm, idx_hbm, noisy_v, out_v, idx_v,
                sem_in, sem_out):
    wid = lax.axis_index("s") * 2 + lax.axis_index("c")
    base = wid * CHUNK

    cp_in0 = pltpu.async_copy(
        noisy_hbm.at[pl.ds(base, HALF)], noisy_v.at[pl.ds(0, HALF)], sem_in)
    cp_in1 = pltpu.async_copy(
        noisy_hbm.at[pl.ds(base + HALF, HALF)], noisy_v.at[pl.ds(HALF, HALF)],
        sem_in)

    lanes = lax.iota(jnp.int32, L)
    zero_i = jnp.full((L,), 0, jnp.int32)
    one_f = jnp.full((L,), 1.0, jnp.float32)
    zero_f = jnp.full((L,), 0.0, jnp.float32)

    def group(g):
        toks = lanes + jnp.full((L,), g * L, jnp.int32)
        mask = jnp.full((L,), E - 1, jnp.int32)
        m1 = plsc.load_gather(noisy_v, [toks, lanes])
        i1 = lanes
        m2 = jnp.full((L,), -jnp.inf, jnp.float32)
        i2 = zero_i
        for e in range(1, E):
            es = (jnp.full((L,), e, jnp.int32) + lanes) & mask
            ve = plsc.load_gather(noisy_v, [toks, es])
            gt1 = ve > m1
            gt2 = ve > m2
            m2 = jnp.where(gt1, m1, jnp.where(gt2, ve, m2))
            i2 = jnp.where(gt1, i1, jnp.where(gt2, es, i2))
            m1 = jnp.where(gt1, ve, m1)
            i1 = jnp.where(gt1, es, i1)
        e2 = jnp.exp(m2 - m1)
        p1 = one_f / (one_f + e2)
        p2 = e2 * p1
        for e in range(E):
            es = (jnp.full((L,), e, jnp.int32) + lanes) & mask
            col = jnp.where(i1 == es, p1, jnp.where(i2 == es, p2, zero_f))
            plsc.store_scatter(out_v, [toks, es], col)
        plsc.store_scatter(idx_v, [toks, zero_i], i1)
        plsc.store_scatter(idx_v, [toks, zero_i + 1], i2)

    cp_in0.wait()
    lax.fori_loop(0, GROUPS // 2, lambda g, c: (group(g), c)[1], 0)
    cp_out0 = pltpu.async_copy(
        out_v.at[pl.ds(0, HALF)], sparse_hbm.at[pl.ds(base, HALF)], sem_out)
    cp_in1.wait()
    lax.fori_loop(GROUPS // 2, GROUPS, lambda g, c: (group(g), c)[1], 0)
    cp_out0.wait()
    pltpu.sync_copy(out_v.at[pl.ds(HALF, HALF)],
                    sparse_hbm.at[pl.ds(base + HALF, HALF)])
    pltpu.sync_copy(idx_v, idx_hbm.at[pl.ds(base, CHUNK)])


@jax.jit
def kernel(x, Wg, bg, Wn, bn, eps):
    noisy = pl.pallas_call(
        _noisy_body,
        grid=(N // BLOCK,),
        in_specs=[
            pl.BlockSpec((BLOCK, D), lambda i: (i, 0)),
            pl.BlockSpec((D, 2 * E), lambda i: (0, 0)),
            pl.BlockSpec((1, 2 * E), lambda i: (0, 0)),
            pl.BlockSpec((BLOCK, E), lambda i: (i, 0)),
        ],
        out_specs=pl.BlockSpec((BLOCK, E), lambda i: (i, 0)),
        out_shape=jax.ShapeDtypeStruct((N, E), jnp.float32),
    )(x, jnp.concatenate([Wg, Wn], axis=1),
      jnp.concatenate([bg, bn]).reshape(1, 2 * E), eps)

    route = pl.kernel(
        _route_body,
        out_type=(
            jax.ShapeDtypeStruct((N, E), jnp.float32),
            jax.ShapeDtypeStruct((N, TOP_K), jnp.int32),
        ),
        mesh=plsc.VectorSubcoreMesh(core_axis_name="c", subcore_axis_name="s"),
        compiler_params=pltpu.CompilerParams(
            needs_layout_passes=False,
            disable_bounds_checks=True,
            disable_semaphore_checks=True,
        ),
        scratch_types=[
            pltpu.VMEM((CHUNK, E), jnp.float32),
            pltpu.VMEM((CHUNK, E), jnp.float32),
            pltpu.VMEM((CHUNK, TOP_K), jnp.int32),
            pltpu.SemaphoreType.DMA,
            pltpu.SemaphoreType.DMA,
        ],
    )
    sparse, idx = route(noisy)
    return sparse, idx
